# Initial kernel scaffold; baseline (speedup 1.0000x reference)
import jax
import jax.numpy as jnp
from jax.experimental import pallas as pl


def kernel(indices, table, W):
    raise NotImplementedError("write your pallas kernel here")


# speedup vs baseline: 2.0742x; 2.0742x over previous
import functools

import jax
import jax.numpy as jnp
from jax import lax
from jax.experimental import pallas as pl
from jax.experimental.pallas import tpu as pltpu
from jax.experimental.pallas import tpu_sc as plsc

NC = 2
NS = 16
NW = NC * NS
CHUNK = 128


def _sc_gather(table, idx3):
    n_chunks = idx3.shape[1]
    rows_per_w = n_chunks * CHUNK
    total = NW * rows_per_w
    d = table.shape[1]
    mesh = plsc.VectorSubcoreMesh(core_axis_name="c", subcore_axis_name="s")

    @functools.partial(
        pl.kernel,
        mesh=mesh,
        compiler_params=pltpu.CompilerParams(use_tc_tiling_on_sc=False),
        out_type=jax.ShapeDtypeStruct((total, d), jnp.float32),
        scratch_types=[
            pltpu.VMEM((n_chunks, CHUNK), jnp.int32),
            pltpu.VMEM((CHUNK, d), jnp.float32),
            pltpu.SemaphoreType.DMA,
        ],
    )
    def k(idx_hbm, table_hbm, out_hbm, idx_v, rows_v, gsem):
        wid = lax.axis_index("c") * NS + lax.axis_index("s")
        base = wid * rows_per_w
        pltpu.sync_copy(idx_hbm.at[wid], idx_v)

        def body(j, carry):
            pltpu.async_copy(table_hbm.at[idx_v.at[j]], rows_v, gsem).wait()
            pltpu.sync_copy(rows_v, out_hbm.at[pl.ds(base + j * CHUNK, CHUNK)])
            return carry

        lax.fori_loop(0, n_chunks, body, 0)

    return k


def _tc_project(emb, wt):
    n, d = emb.shape
    e = wt.shape[1]
    br = 2048
    assert n % br == 0

    def mm(emb_ref, wt_ref, out_ref):
        out_ref[...] = jnp.dot(
            emb_ref[...], wt_ref[...], preferred_element_type=jnp.float32
        )

    return pl.pallas_call(
        mm,
        grid=(n // br,),
        in_specs=[
            pl.BlockSpec((br, d), lambda i: (i, 0)),
            pl.BlockSpec((d, e), lambda i: (0, 0)),
        ],
        out_specs=pl.BlockSpec((br, e), lambda i: (i, 0)),
        out_shape=jax.ShapeDtypeStruct((n, e), jnp.float32),
        compiler_params=pltpu.CompilerParams(
            dimension_semantics=("arbitrary",),
        ),
    )(emb, wt)


def kernel(indices, table, W):
    b, l = indices.shape
    total = b * l
    assert total % (NW * CHUNK) == 0
    n_chunks = total // (NW * CHUNK)
    idx3 = indices.reshape(NW, n_chunks, CHUNK).astype(jnp.int32)
    emb = _sc_gather(table, idx3)(idx3, table)
    out = _tc_project(emb, W.T)
    return out.reshape(b, l, W.shape[0])


# speedup vs baseline: 2.2228x; 1.0717x over previous
import functools

import jax
import jax.numpy as jnp
from jax import lax
from jax.experimental import pallas as pl
from jax.experimental.pallas import tpu as pltpu
from jax.experimental.pallas import tpu_sc as plsc

NC = 2
NS = 16
NW = NC * NS
CHUNK = 128


def _sc_gather(table, idx3):
    n_chunks = idx3.shape[1]
    rows_per_w = n_chunks * CHUNK
    total = NW * rows_per_w
    d = table.shape[1]
    mesh = plsc.VectorSubcoreMesh(core_axis_name="c", subcore_axis_name="s")
    nbuf = 5
    assert n_chunks % nbuf == 0 and n_chunks >= nbuf

    @functools.partial(
        pl.kernel,
        mesh=mesh,
        compiler_params=pltpu.CompilerParams(use_tc_tiling_on_sc=False),
        out_type=jax.ShapeDtypeStruct((total, d), jnp.float32),
        scratch_types=[
            pltpu.VMEM((n_chunks, CHUNK), jnp.int32),
            pltpu.VMEM((nbuf, CHUNK, d), jnp.float32),
        ]
        + [pltpu.SemaphoreType.DMA] * (2 * nbuf),
    )
    def k(idx_hbm, table_hbm, out_hbm, idx_v, rows_v, *sems):
        gsems, osems = sems[:nbuf], sems[nbuf:]
        wid = lax.axis_index("c") * NS + lax.axis_index("s")
        base = wid * rows_per_w
        pltpu.sync_copy(idx_hbm.at[wid], idx_v)

        def gather(j, b):
            pltpu.async_copy(table_hbm.at[idx_v.at[j]], rows_v.at[b], gsems[b])

        def gather_wait(j, b):
            pltpu.make_async_copy(
                table_hbm.at[idx_v.at[j]], rows_v.at[b], gsems[b]
            ).wait()

        def out_start(j, b):
            pltpu.async_copy(
                rows_v.at[b], out_hbm.at[pl.ds(base + j * CHUNK, CHUNK)], osems[b]
            )

        def out_wait(j, b):
            pltpu.make_async_copy(
                rows_v.at[b], out_hbm.at[pl.ds(base + j * CHUNK, CHUNK)], osems[b]
            ).wait()

        for b in range(nbuf):
            gather(b, b)

        def group(g, carry):
            for b in range(nbuf):
                j = g * nbuf + b
                gather_wait(j, b)
                out_start(j, b)

                @pl.when(j + nbuf < n_chunks)
                def _():
                    out_wait(j, b)
                    gather(j + nbuf, b)

            return carry

        lax.fori_loop(0, n_chunks // nbuf, group, 0)
        for b in range(nbuf):
            out_wait(n_chunks - nbuf + b, b)

    return k


def _tc_project(emb, wt):
    n, d = emb.shape
    e = wt.shape[1]
    br = 2048
    assert n % br == 0

    def mm(emb_ref, wt_ref, out_ref):
        out_ref[...] = jnp.dot(
            emb_ref[...], wt_ref[...], preferred_element_type=jnp.float32
        )

    return pl.pallas_call(
        mm,
        grid=(n // br,),
        in_specs=[
            pl.BlockSpec((br, d), lambda i: (i, 0)),
            pl.BlockSpec((d, e), lambda i: (0, 0)),
        ],
        out_specs=pl.BlockSpec((br, e), lambda i: (i, 0)),
        out_shape=jax.ShapeDtypeStruct((n, e), jnp.float32),
        compiler_params=pltpu.CompilerParams(
            dimension_semantics=("arbitrary",),
        ),
    )(emb, wt)


def kernel(indices, table, W):
    b, l = indices.shape
    total = b * l
    assert total % (NW * CHUNK) == 0
    n_chunks = total // (NW * CHUNK)
    idx3 = indices.reshape(NW, n_chunks, CHUNK).astype(jnp.int32)
    emb = _sc_gather(table, idx3)(idx3, table)
    out = _tc_project(emb, W.T)
    return out.reshape(b, l, W.shape[0])


# speedup vs baseline: 2.9161x; 1.3119x over previous
import functools

import jax
import jax.numpy as jnp
from jax import lax
from jax.experimental import pallas as pl
from jax.experimental.pallas import tpu as pltpu
from jax.experimental.pallas import tpu_sc as plsc

NC = 2
NS = 16
NW = NC * NS
CHUNK = 128


def _sc_gather(proj, idx3):
    n_chunks = idx3.shape[1]
    rows_per_w = n_chunks * CHUNK
    total = NW * rows_per_w
    e = proj.shape[1]
    mesh = plsc.VectorSubcoreMesh(core_axis_name="c", subcore_axis_name="s")
    nbuf = 5
    assert n_chunks % nbuf == 0 and n_chunks >= nbuf

    @functools.partial(
        pl.kernel,
        mesh=mesh,
        out_type=jax.ShapeDtypeStruct((total, e), jnp.float32),
        scratch_types=[
            pltpu.VMEM((n_chunks, CHUNK), jnp.int32),
            pltpu.VMEM((nbuf, CHUNK, e), jnp.float32),
        ]
        + [pltpu.SemaphoreType.DMA] * (2 * nbuf),
    )
    def k(idx_hbm, proj_hbm, out_hbm, idx_v, rows_v, *sems):
        gsems, osems = sems[:nbuf], sems[nbuf:]
        wid = lax.axis_index("c") * NS + lax.axis_index("s")
        base = wid * rows_per_w
        pltpu.sync_copy(idx_hbm.at[wid], idx_v)

        def gather(j, b):
            pltpu.async_copy(proj_hbm.at[idx_v.at[j]], rows_v.at[b], gsems[b])

        def gather_wait(j, b):
            pltpu.make_async_copy(
                proj_hbm.at[idx_v.at[j]], rows_v.at[b], gsems[b]
            ).wait()

        def out_start(j, b):
            pltpu.async_copy(
                rows_v.at[b], out_hbm.at[pl.ds(base + j * CHUNK, CHUNK)], osems[b]
            )

        def out_wait(j, b):
            pltpu.make_async_copy(
                rows_v.at[b], out_hbm.at[pl.ds(base + j * CHUNK, CHUNK)], osems[b]
            ).wait()

        for b in range(nbuf):
            gather(b, b)

        def group(g, carry):
            for b in range(nbuf):
                j = g * nbuf + b
                gather_wait(j, b)
                out_start(j, b)

                @pl.when(j + nbuf < n_chunks)
                def _():
                    out_wait(j, b)
                    gather(j + nbuf, b)

            return carry

        lax.fori_loop(0, n_chunks // nbuf, group, 0)
        for b in range(nbuf):
            out_wait(n_chunks - nbuf + b, b)

    return k


def _tc_project(table, wt):
    v, d = table.shape
    e = wt.shape[1]
    br = 2000
    assert v % br == 0

    def mm(t_ref, wt_ref, out_ref):
        out_ref[...] = jnp.dot(
            t_ref[...], wt_ref[...], preferred_element_type=jnp.float32
        )

    return pl.pallas_call(
        mm,
        grid=(v // br,),
        in_specs=[
            pl.BlockSpec((br, d), lambda i: (i, 0)),
            pl.BlockSpec((d, e), lambda i: (0, 0)),
        ],
        out_specs=pl.BlockSpec((br, e), lambda i: (i, 0)),
        out_shape=jax.ShapeDtypeStruct((v, e), jnp.float32),
        compiler_params=pltpu.CompilerParams(
            dimension_semantics=("arbitrary",),
        ),
    )(table, wt)


def kernel(indices, table, W):
    b, l = indices.shape
    total = b * l
    assert total % (NW * CHUNK) == 0
    n_chunks = total // (NW * CHUNK)
    idx3 = indices.reshape(NW, n_chunks, CHUNK).astype(jnp.int32)
    proj = _tc_project(table, W.T)
    out = _sc_gather(proj, idx3)(idx3, proj)
    return out.reshape(b, l, W.shape[0])


# speedup vs baseline: 4.3571x; 1.4942x over previous
import functools

import jax
import jax.numpy as jnp
from jax import lax
from jax.experimental import pallas as pl
from jax.experimental.pallas import tpu as pltpu
from jax.experimental.pallas import tpu_sc as plsc

NC = 2
NS = 16
NW = NC * NS
MAXQ = 8


def _sc_gather(proj, indices):
    bsz, hist = indices.shape
    e = proj.shape[1]
    assert bsz % NW == 0
    b_per_w = bsz // NW
    mesh = plsc.VectorSubcoreMesh(core_axis_name="c", subcore_axis_name="s")

    nbuf = 4
    assert b_per_w % nbuf == 0 and b_per_w >= nbuf

    @functools.partial(
        pl.kernel,
        mesh=mesh,
        out_type=jax.ShapeDtypeStruct((bsz, hist, e), jnp.float32),
        scratch_types=[
            pltpu.VMEM((b_per_w, hist), jnp.int32),
            pltpu.VMEM((nbuf, hist, e), jnp.float32),
        ]
        + [pltpu.SemaphoreType.DMA] * (2 * nbuf),
    )
    def k(idx_hbm, proj_hbm, out_hbm, idx_v, rows_v, *sems):
        gsems, osems = sems[:nbuf], sems[nbuf:]
        wid = lax.axis_index("c") * NS + lax.axis_index("s")
        base = wid * b_per_w
        pltpu.sync_copy(idx_hbm.at[pl.ds(base, b_per_w)], idx_v)

        def gather(j, b):
            pltpu.async_copy(proj_hbm.at[idx_v.at[j]], rows_v.at[b], gsems[b])

        def gather_wait(j, b):
            pltpu.make_async_copy(
                proj_hbm.at[idx_v.at[j]], rows_v.at[b], gsems[b]
            ).wait()

        def out_start(j, b):
            pltpu.async_copy(rows_v.at[b], out_hbm.at[base + j], osems[b])

        def out_wait(j, b):
            pltpu.make_async_copy(
                rows_v.at[b], out_hbm.at[base + j], osems[b]
            ).wait()

        for b in range(nbuf):
            gather(b, b)

        def group(g, carry):
            for b in range(nbuf):
                j = g * nbuf + b
                gather_wait(j, b)
                out_start(j, b)

                @pl.when(j + nbuf < b_per_w)
                def _():
                    out_wait(j, b)
                    gather(j + nbuf, b)

            return carry

        lax.fori_loop(0, b_per_w // nbuf, group, 0)
        for b in range(nbuf):
            out_wait(b_per_w - nbuf + b, b)

    return k


def _tc_project(table, wt):
    v, d = table.shape
    e = wt.shape[1]
    br = 2000
    assert v % br == 0

    def mm(t_ref, wt_ref, out_ref):
        out_ref[...] = jnp.dot(
            t_ref[...], wt_ref[...], preferred_element_type=jnp.float32
        )

    return pl.pallas_call(
        mm,
        grid=(v // br,),
        in_specs=[
            pl.BlockSpec((br, d), lambda i: (i, 0)),
            pl.BlockSpec((d, e), lambda i: (0, 0)),
        ],
        out_specs=pl.BlockSpec((br, e), lambda i: (i, 0)),
        out_shape=jax.ShapeDtypeStruct((v, e), jnp.float32),
        compiler_params=pltpu.CompilerParams(
            dimension_semantics=("arbitrary",),
        ),
    )(table, wt)


def kernel(indices, table, W):
    proj = _tc_project(table, W.T)
    return _sc_gather(proj, indices)(indices.astype(jnp.int32), proj)


# speedup vs baseline: 9.4820x; 2.1762x over previous
import functools

import jax
import jax.numpy as jnp
from jax import lax
from jax.experimental import pallas as pl
from jax.experimental.pallas import tpu as pltpu
from jax.experimental.pallas import tpu_sc as plsc

NC = 2
NS = 16
NW = NC * NS
BCHUNK = 128


def _sc_gather(proj, idx_t):
    hist, bsz = idx_t.shape
    e = proj.shape[1]
    assert bsz == NW * BCHUNK
    n_chunks = hist
    mesh = plsc.VectorSubcoreMesh(core_axis_name="c", subcore_axis_name="s")
    nbuf = 5
    assert n_chunks % nbuf == 0 and n_chunks >= nbuf

    @functools.partial(
        pl.kernel,
        mesh=mesh,
        out_type=jax.ShapeDtypeStruct((hist, bsz, e), jnp.float32),
        scratch_types=[
            pltpu.VMEM((n_chunks, BCHUNK), jnp.int32),
            pltpu.VMEM((nbuf, BCHUNK, e), jnp.float32),
        ]
        + [pltpu.SemaphoreType.DMA] * (2 * nbuf),
    )
    def k(idx_hbm, proj_hbm, out_hbm, idx_v, rows_v, *sems):
        gsems, osems = sems[:nbuf], sems[nbuf:]
        wid = lax.axis_index("c") * NS + lax.axis_index("s")
        base = wid * BCHUNK
        pltpu.sync_copy(idx_hbm.at[:, pl.ds(base, BCHUNK)], idx_v)

        def gather(j, b):
            pltpu.async_copy(proj_hbm.at[idx_v.at[j]], rows_v.at[b], gsems[b])

        def gather_wait(j, b):
            pltpu.make_async_copy(
                proj_hbm.at[idx_v.at[j]], rows_v.at[b], gsems[b]
            ).wait()

        def out_start(j, b):
            pltpu.async_copy(
                rows_v.at[b], out_hbm.at[j, pl.ds(base, BCHUNK)], osems[b]
            )

        def out_wait(j, b):
            pltpu.make_async_copy(
                rows_v.at[b], out_hbm.at[j, pl.ds(base, BCHUNK)], osems[b]
            ).wait()

        for b in range(nbuf):
            gather(b, b)

        def group(g, carry):
            for b in range(nbuf):
                j = g * nbuf + b
                gather_wait(j, b)
                out_start(j, b)

                @pl.when(j + nbuf < n_chunks)
                def _():
                    out_wait(j, b)
                    gather(j + nbuf, b)

            return carry

        lax.fori_loop(0, n_chunks // nbuf, group, 0)
        for b in range(nbuf):
            out_wait(n_chunks - nbuf + b, b)

    return k


def _tc_project(table_t, wt):
    d, v = table_t.shape
    e = wt.shape[1]
    bv = 8192
    grid = (v + bv - 1) // bv

    def mm(t_ref, wt_ref, out_ref):
        out_ref[...] = jax.lax.dot_general(
            t_ref[...],
            wt_ref[...],
            dimension_numbers=(((0,), (0,)), ((), ())),
            preferred_element_type=jnp.float32,
        )

    return pl.pallas_call(
        mm,
        grid=(grid,),
        in_specs=[
            pl.BlockSpec((d, bv), lambda i: (0, i)),
            pl.BlockSpec((d, e), lambda i: (0, 0)),
        ],
        out_specs=pl.BlockSpec((bv, e), lambda i: (i, 0)),
        out_shape=jax.ShapeDtypeStruct((v, e), jnp.float32),
        compiler_params=pltpu.CompilerParams(
            dimension_semantics=("arbitrary",),
        ),
    )(table_t, wt)


def kernel(indices, table, W):
    proj = _tc_project(table.T, W.T)
    out_t = _sc_gather(proj, indices.T)(indices.T.astype(jnp.int32), proj)
    return out_t.transpose(1, 0, 2)


# speedup vs baseline: 9.6252x; 1.0151x over previous
import functools

import jax
import jax.numpy as jnp
from jax import lax
from jax.experimental import pallas as pl
from jax.experimental.pallas import tpu as pltpu
from jax.experimental.pallas import tpu_sc as plsc

NC = 2
NS = 16
NW = NC * NS
BCHUNK = 128


def _sc_gather(proj, idx_t):
    hist, bsz = idx_t.shape
    e = proj.shape[1]
    assert bsz == NW * BCHUNK
    n_chunks = hist
    mesh = plsc.VectorSubcoreMesh(core_axis_name="c", subcore_axis_name="s")
    nbuf = 5
    assert n_chunks % nbuf == 0 and n_chunks >= nbuf

    @functools.partial(
        pl.kernel,
        mesh=mesh,
        out_type=jax.ShapeDtypeStruct((hist, bsz, e), jnp.float32),
        scratch_types=[
            pltpu.VMEM((n_chunks, BCHUNK), jnp.int32),
            pltpu.VMEM((nbuf, BCHUNK, e), jnp.float32),
        ]
        + [pltpu.SemaphoreType.DMA] * (2 * nbuf),
    )
    def k(idx_hbm, proj_hbm, out_hbm, idx_v, rows_v, *sems):
        gsems, osems = sems[:nbuf], sems[nbuf:]
        wid = lax.axis_index("c") * NS + lax.axis_index("s")
        base = wid * BCHUNK
        pltpu.sync_copy(idx_hbm.at[:, pl.ds(base, BCHUNK)], idx_v)

        def gather(j, b):
            pltpu.async_copy(proj_hbm.at[idx_v.at[j]], rows_v.at[b], gsems[b])

        def gather_wait(j, b):
            pltpu.make_async_copy(
                proj_hbm.at[idx_v.at[j]], rows_v.at[b], gsems[b]
            ).wait()

        def out_start(j, b):
            pltpu.async_copy(
                rows_v.at[b], out_hbm.at[j, pl.ds(base, BCHUNK)], osems[b]
            )

        def out_wait(j, b):
            pltpu.make_async_copy(
                rows_v.at[b], out_hbm.at[j, pl.ds(base, BCHUNK)], osems[b]
            ).wait()

        for b in range(nbuf):
            gather(b, b)

        def group(g, carry):
            for b in range(nbuf):
                j = g * nbuf + b
                gather_wait(j, b)
                out_start(j, b)

                @pl.when(j + nbuf < n_chunks)
                def _():
                    out_wait(j, b)
                    gather(j + nbuf, b)

            return carry

        lax.fori_loop(0, n_chunks // nbuf, group, 0)
        for b in range(nbuf):
            out_wait(n_chunks - nbuf + b, b)

    return k


def _tc_project(table_t, wt):
    d, v = table_t.shape
    e = wt.shape[1]
    bv = 16384
    grid = (v + bv - 1) // bv

    def mm(t_ref, wt_ref, out_ref):
        out_ref[...] = jax.lax.dot_general(
            t_ref[...],
            wt_ref[...],
            dimension_numbers=(((0,), (0,)), ((), ())),
            preferred_element_type=jnp.float32,
        )

    return pl.pallas_call(
        mm,
        grid=(grid,),
        in_specs=[
            pl.BlockSpec((d, bv), lambda i: (0, i)),
            pl.BlockSpec((d, e), lambda i: (0, 0)),
        ],
        out_specs=pl.BlockSpec((bv, e), lambda i: (i, 0)),
        out_shape=jax.ShapeDtypeStruct((v, e), jnp.float32),
        compiler_params=pltpu.CompilerParams(
            dimension_semantics=("arbitrary",),
        ),
    )(table_t, wt)


def kernel(indices, table, W):
    proj = _tc_project(table.T, W.T)
    out_t = _sc_gather(proj, indices.T)(indices.T.astype(jnp.int32), proj)
    return out_t.transpose(1, 0, 2)


# speedup vs baseline: 9.6941x; 1.0072x over previous
import functools

import jax
import jax.numpy as jnp
from jax import lax
from jax.experimental import pallas as pl
from jax.experimental.pallas import tpu as pltpu
from jax.experimental.pallas import tpu_sc as plsc

NC = 2
NS = 16
NW = NC * NS
BCHUNK = 128


def _sc_gather(proj, idx_t):
    hist, bsz = idx_t.shape
    e = proj.shape[1]
    assert bsz == NW * BCHUNK
    n_chunks = hist
    mesh = plsc.VectorSubcoreMesh(core_axis_name="c", subcore_axis_name="s")
    nbuf = 7
    n_groups = (n_chunks + nbuf - 1) // nbuf
    assert n_chunks >= nbuf

    @functools.partial(
        pl.kernel,
        mesh=mesh,
        out_type=jax.ShapeDtypeStruct((hist, bsz, e), jnp.float32),
        scratch_types=[
            pltpu.VMEM((n_chunks, BCHUNK), jnp.int32),
            pltpu.VMEM((nbuf, BCHUNK, e), jnp.float32),
        ]
        + [pltpu.SemaphoreType.DMA] * (2 * nbuf),
    )
    def k(idx_hbm, proj_hbm, out_hbm, idx_v, rows_v, *sems):
        gsems, osems = sems[:nbuf], sems[nbuf:]
        wid = lax.axis_index("c") * NS + lax.axis_index("s")
        base = wid * BCHUNK
        pltpu.sync_copy(idx_hbm.at[:, pl.ds(base, BCHUNK)], idx_v)

        def gather(j, b):
            pltpu.async_copy(proj_hbm.at[idx_v.at[j]], rows_v.at[b], gsems[b])

        def gather_wait(j, b):
            pltpu.make_async_copy(
                proj_hbm.at[idx_v.at[j]], rows_v.at[b], gsems[b]
            ).wait()

        def out_start(j, b):
            pltpu.async_copy(
                rows_v.at[b], out_hbm.at[j, pl.ds(base, BCHUNK)], osems[b]
            )

        def out_wait(j, b):
            pltpu.make_async_copy(
                rows_v.at[b], out_hbm.at[j, pl.ds(base, BCHUNK)], osems[b]
            ).wait()

        for b in range(nbuf):
            gather(b, b)

        def group(g, carry):
            for b in range(nbuf):
                j = g * nbuf + b

                @pl.when(j < n_chunks)
                def _():
                    gather_wait(j, b)
                    out_start(j, b)

                    @pl.when(j + nbuf < n_chunks)
                    def _():
                        out_wait(j, b)
                        gather(j + nbuf, b)

            return carry

        lax.fori_loop(0, n_groups, group, 0)
        for j in range(n_chunks - nbuf, n_chunks):
            out_wait(j, j % nbuf)

    return k


def _tc_project(table_t, wt):
    d, v = table_t.shape
    e = wt.shape[1]
    bv = 16384
    grid = (v + bv - 1) // bv

    def mm(t_ref, wt_ref, out_ref):
        out_ref[...] = jax.lax.dot_general(
            t_ref[...],
            wt_ref[...],
            dimension_numbers=(((0,), (0,)), ((), ())),
            preferred_element_type=jnp.float32,
        )

    return pl.pallas_call(
        mm,
        grid=(grid,),
        in_specs=[
            pl.BlockSpec((d, bv), lambda i: (0, i)),
            pl.BlockSpec((d, e), lambda i: (0, 0)),
        ],
        out_specs=pl.BlockSpec((bv, e), lambda i: (i, 0)),
        out_shape=jax.ShapeDtypeStruct((v, e), jnp.float32),
        compiler_params=pltpu.CompilerParams(
            dimension_semantics=("arbitrary",),
        ),
    )(table_t, wt)


def kernel(indices, table, W):
    proj = _tc_project(table.T, W.T)
    out_t = _sc_gather(proj, indices.T)(indices.T.astype(jnp.int32), proj)
    return out_t.transpose(1, 0, 2)
